# baseline (speedup 1.0000x reference)
import functools

import jax
import jax.numpy as jnp
from jax import lax
from jax.experimental import pallas as pl
from jax.experimental.pallas import tpu as pltpu
from jax.experimental.pallas import tpu_sc as plsc

_NC = 2
_NS = 16
_NW = _NC * _NS

_SUB = 100
_ROWS_PER_CHUNK = 4
_NBUF = 2


@functools.partial(jax.jit, static_argnums=(2, 3, 4))
def _gather(idx2d, table, N, J, D):
    nsub = _ROWS_PER_CHUNK * J // _SUB
    i_per_w = N // _NW
    n_chunks = i_per_w // _ROWS_PER_CHUNK
    npair = n_chunks // _NBUF

    mesh = plsc.VectorSubcoreMesh(
        core_axis_name="c", subcore_axis_name="s",
        num_cores=_NC, num_subcores=_NS)

    @functools.partial(
        pl.kernel,
        out_type=jax.ShapeDtypeStruct((N, J, D), jnp.float32),
        mesh=mesh,
        scratch_types=[
            pltpu.VMEM((_NBUF, nsub, _SUB), jnp.int32),
            pltpu.VMEM((_NBUF, _ROWS_PER_CHUNK, J, D), jnp.float32),
            [pltpu.SemaphoreType.DMA] * _NBUF,
            [pltpu.SemaphoreType.DMA] * _NBUF,
            [pltpu.SemaphoreType.DMA] * _NBUF,
        ],
        compiler_params=pltpu.CompilerParams(use_tc_tiling_on_sc=False),
    )
    def k(idx_hbm, table_hbm, out_hbm, idx_v, rows_v, semi, semg, semo):
        wid = lax.axis_index("s") * _NC + lax.axis_index("c")
        idxrow0 = wid * (i_per_w * J // _SUB)
        row0 = wid * i_per_w

        def start_idx(b, g):
            pltpu.async_copy(
                idx_hbm.at[pl.ds(idxrow0 + g * nsub, nsub)],
                idx_v.at[b], semi[b])

        def wait_idx(b):
            pltpu.make_async_copy(
                idx_hbm.at[pl.ds(idxrow0, nsub)],
                idx_v.at[b], semi[b]).wait()

        def run_gather(b):
            waits = []
            for j in range(nsub):
                waits.append(pltpu.async_copy(
                    table_hbm.at[idx_v.at[b, j]],
                    rows_v.at[b, (j * _SUB) // J,
                              pl.ds((j * _SUB) % J, _SUB)],
                    semg[b]))
            for w in waits:
                w.wait()

        def start_out(b, g):
            pltpu.async_copy(
                rows_v.at[b],
                out_hbm.at[pl.ds(row0 + g * _ROWS_PER_CHUNK,
                                 _ROWS_PER_CHUNK)],
                semo[b])

        def wait_out(b):
            pltpu.make_async_copy(
                rows_v.at[b],
                out_hbm.at[pl.ds(row0, _ROWS_PER_CHUNK)],
                semo[b]).wait()

        for b in range(_NBUF):
            start_idx(b, b)
        for b in range(_NBUF):
            wait_idx(b)
            run_gather(b)
            start_idx(b, b + _NBUF)
            start_out(b, b)

        @pl.loop(1, npair - 1)
        def _pair(p):
            for b in range(_NBUF):
                g = p * _NBUF + b
                wait_idx(b)
                wait_out(b)
                run_gather(b)
                start_idx(b, g + _NBUF)
                start_out(b, g)

        for b in range(_NBUF):
            g = n_chunks - _NBUF + b
            wait_idx(b)
            wait_out(b)
            run_gather(b)
            start_out(b, g)
        for b in range(_NBUF):
            wait_out(b)

    return k(idx2d, table)


def kernel(x, W):
    N, J = x.shape
    D = W.shape[1]
    idx2d = x.reshape(N * J // _SUB, _SUB).astype(jnp.int32)
    return _gather(idx2d, W, N, J, D)


# speedup vs baseline: 1.0002x; 1.0002x over previous
import functools

import jax
import jax.numpy as jnp
from jax import lax
from jax.experimental import pallas as pl
from jax.experimental.pallas import tpu as pltpu
from jax.experimental.pallas import tpu_sc as plsc

_NC = 2
_NS = 16
_NW = _NC * _NS

_SUB = 100
_ROWS_PER_CHUNK = 4
_NBUF = 2


def _gather(idx2d, table, N, J, D):
    nsub = _ROWS_PER_CHUNK * J // _SUB
    i_per_w = N // _NW
    n_chunks = i_per_w // _ROWS_PER_CHUNK
    npair = n_chunks // _NBUF

    mesh = plsc.VectorSubcoreMesh(
        core_axis_name="c", subcore_axis_name="s",
        num_cores=_NC, num_subcores=_NS)

    @functools.partial(
        pl.kernel,
        out_type=jax.ShapeDtypeStruct((N, J, D), jnp.float32),
        mesh=mesh,
        scratch_types=[
            pltpu.VMEM((_NBUF, nsub, _SUB), jnp.int32),
            pltpu.VMEM((_NBUF, _ROWS_PER_CHUNK, J, D), jnp.float32),
            [pltpu.SemaphoreType.DMA] * _NBUF,
            [pltpu.SemaphoreType.DMA] * _NBUF,
            [pltpu.SemaphoreType.DMA] * _NBUF,
        ],
        compiler_params=pltpu.CompilerParams(use_tc_tiling_on_sc=False),
    )
    def k(idx_hbm, table_hbm, out_hbm, idx_v, rows_v, semi, semg, semo):
        wid = lax.axis_index("s") * _NC + lax.axis_index("c")
        idxrow0 = wid * (i_per_w * J // _SUB)
        row0 = wid * i_per_w

        def start_idx(b, g):
            pltpu.async_copy(
                idx_hbm.at[pl.ds(idxrow0 + g * nsub, nsub)],
                idx_v.at[b], semi[b])

        def wait_idx(b):
            pltpu.make_async_copy(
                idx_hbm.at[pl.ds(idxrow0, nsub)],
                idx_v.at[b], semi[b]).wait()

        def run_gather(b):
            waits = []
            for j in range(nsub):
                waits.append(pltpu.async_copy(
                    table_hbm.at[idx_v.at[b, j]],
                    rows_v.at[b, (j * _SUB) // J,
                              pl.ds((j * _SUB) % J, _SUB)],
                    semg[b]))
            for w in waits:
                w.wait()

        def start_out(b, g):
            pltpu.async_copy(
                rows_v.at[b],
                out_hbm.at[pl.ds(row0 + g * _ROWS_PER_CHUNK,
                                 _ROWS_PER_CHUNK)],
                semo[b])

        def wait_out(b):
            pltpu.make_async_copy(
                rows_v.at[b],
                out_hbm.at[pl.ds(row0, _ROWS_PER_CHUNK)],
                semo[b]).wait()

        for b in range(_NBUF):
            start_idx(b, b)
        for b in range(_NBUF):
            wait_idx(b)
            run_gather(b)
            start_idx(b, b + _NBUF)
            start_out(b, b)

        @pl.loop(1, npair - 1)
        def _pair(p):
            for b in range(_NBUF):
                g = p * _NBUF + b
                wait_idx(b)
                wait_out(b)
                run_gather(b)
                start_idx(b, g + _NBUF)
                start_out(b, g)

        for b in range(_NBUF):
            g = n_chunks - _NBUF + b
            wait_idx(b)
            wait_out(b)
            run_gather(b)
            start_out(b, g)
        for b in range(_NBUF):
            wait_out(b)

    return k(idx2d, table)


def kernel(x, W):
    N, J = x.shape
    D = W.shape[1]
    idx2d = x.reshape(N * J // _SUB, _SUB).astype(jnp.int32)
    return _gather(idx2d, W, N, J, D)


# speedup vs baseline: 1.3261x; 1.3258x over previous
import functools

import jax
import jax.numpy as jnp
from jax import lax
from jax.experimental import pallas as pl
from jax.experimental.pallas import tpu as pltpu
from jax.experimental.pallas import tpu_sc as plsc

_NC = 2
_NS = 16
_NW = _NC * _NS

_SUB = 100
_NSUB = 4
_CHUNK = _SUB * _NSUB
_NBUF = 2
_DP = 128


def _gather(idx2d, table, B):
    b_per_w = B // _NW
    n_chunks = b_per_w // _CHUNK
    npair = n_chunks // _NBUF
    idxrows_per_w = b_per_w // _SUB

    mesh = plsc.VectorSubcoreMesh(
        core_axis_name="c", subcore_axis_name="s",
        num_cores=_NC, num_subcores=_NS)

    @functools.partial(
        pl.kernel,
        out_type=jax.ShapeDtypeStruct((B, _DP), jnp.float32),
        mesh=mesh,
        scratch_types=[
            pltpu.VMEM((_NBUF, _NSUB, _SUB), jnp.int32),
            pltpu.VMEM((_NBUF, _CHUNK, _DP), jnp.float32),
            [pltpu.SemaphoreType.DMA] * _NBUF,
            [pltpu.SemaphoreType.DMA] * _NBUF,
            [pltpu.SemaphoreType.DMA] * _NBUF,
        ],
        compiler_params=pltpu.CompilerParams(use_tc_tiling_on_sc=False),
    )
    def k(idx_hbm, table_hbm, out_hbm, idx_v, rows_v, semi, semg, semo):
        wid = lax.axis_index("s") * _NC + lax.axis_index("c")
        idxrow0 = wid * idxrows_per_w
        row0 = wid * b_per_w

        def start_idx(b, g):
            pltpu.async_copy(
                idx_hbm.at[pl.ds(idxrow0 + g * _NSUB, _NSUB)],
                idx_v.at[b], semi[b])

        def wait_idx(b):
            pltpu.make_async_copy(
                idx_hbm.at[pl.ds(idxrow0, _NSUB)],
                idx_v.at[b], semi[b]).wait()

        def run_gather(b):
            waits = []
            for j in range(_NSUB):
                waits.append(pltpu.async_copy(
                    table_hbm.at[idx_v.at[b, j]],
                    rows_v.at[b, pl.ds(j * _SUB, _SUB)],
                    semg[b]))
            for w in waits:
                w.wait()

        def start_out(b, g):
            pltpu.async_copy(
                rows_v.at[b],
                out_hbm.at[pl.ds(row0 + g * _CHUNK, _CHUNK)],
                semo[b])

        def wait_out(b):
            pltpu.make_async_copy(
                rows_v.at[b],
                out_hbm.at[pl.ds(row0, _CHUNK)],
                semo[b]).wait()

        for b in range(_NBUF):
            start_idx(b, b)
        for b in range(_NBUF):
            wait_idx(b)
            run_gather(b)
            start_idx(b, b + _NBUF)
            start_out(b, b)

        @pl.loop(1, npair - 1)
        def _pair(p):
            for b in range(_NBUF):
                g = p * _NBUF + b
                wait_idx(b)
                wait_out(b)
                run_gather(b)
                start_idx(b, g + _NBUF)
                start_out(b, g)

        for b in range(_NBUF):
            g = n_chunks - _NBUF + b
            wait_idx(b)
            wait_out(b)
            run_gather(b)
            start_out(b, g)
        for b in range(_NBUF):
            wait_out(b)

    return k(idx2d, table)


def kernel(x, W):
    N, J = x.shape
    D = W.shape[1]
    B = N * J
    idx2d = x.reshape(B // _SUB, _SUB).astype(jnp.int32)
    W_pad = jnp.pad(W, ((0, 0), (0, _DP - D)))
    out2 = _gather(idx2d, W_pad, B)
    return out2[:, :D].reshape(N, J, D)
